# baseline (speedup 1.0000x reference)
import functools

import numpy as np
import jax
import jax.numpy as jnp
from jax import lax
from jax.experimental import pallas as pl
from jax.experimental.pallas import tpu as pltpu
from jax.experimental.pallas import tpu_sc as plsc

_NB = 100
_NP = 4194304
_NN = 16777216
_NC, _NS, _L = 2, 16, 16
_NW = _NC * _NS
_CH = 32768
_HP = 112
_STEP = float(np.float32(1.0) / np.float32(_NB))

_MESH = plsc.VectorSubcoreMesh(
    core_axis_name="c", subcore_axis_name="s", num_cores=_NC, num_subcores=_NS
)


def _wid():
    return lax.axis_index("s") * _NC + lax.axis_index("c")


def _perm(v, idx):
    return lax.gather(
        v,
        idx[:, None],
        lax.GatherDimensionNumbers(
            offset_dims=(), collapsed_slice_dims=(0,), start_index_map=(0,)
        ),
        (1,),
        mode=lax.GatherScatterMode.PROMISE_IN_BOUNDS,
    )


def _bfly_sum(v, lane):
    for k in (1, 2, 4, 8):
        v = v + _perm(v, lane ^ k)
    return v


def _scan_chunks(hbm, base, nch, b0, b1, s0, s1, body, carry):
    pltpu.async_copy(hbm.at[pl.ds(base, _CH)], b0, s0)
    pltpu.async_copy(hbm.at[pl.ds(base + _CH, _CH)], b1, s1)

    def pair(p, c):
        off = base + 2 * p * _CH
        pltpu.make_async_copy(hbm.at[pl.ds(0, _CH)], b0, s0).wait()
        c = body(b0, c)

        @pl.when(2 * p + 2 < nch)
        def _():
            pltpu.async_copy(hbm.at[pl.ds(off + 2 * _CH, _CH)], b0, s0)

        pltpu.make_async_copy(hbm.at[pl.ds(0, _CH)], b1, s1).wait()
        c = body(b1, c)

        @pl.when(2 * p + 3 < nch)
        def _():
            pltpu.async_copy(hbm.at[pl.ds(off + 3 * _CH, _CH)], b1, s1)

        return c

    return lax.fori_loop(0, nch // 2, pair, carry)


@functools.partial(
    pl.kernel,
    out_type=jax.ShapeDtypeStruct((_NW * 64,), jnp.float32),
    mesh=_MESH,
    compiler_params=pltpu.CompilerParams(needs_layout_passes=False),
    scratch_types=[
        pltpu.VMEM((_CH,), jnp.float32),
        pltpu.VMEM((_CH,), jnp.float32),
        pltpu.VMEM((64,), jnp.float32),
        pltpu.SemaphoreType.DMA,
        pltpu.SemaphoreType.DMA,
    ],
)
def _minmax_k(pos_hbm, neg_hbm, out_hbm, b0, b1, ob, s0, s1):
    w = _wid()

    def mm(buf, c):
        def step(i, c2):
            mn, nmx = c2
            v = buf[pl.ds(i * _L, _L)]
            return jnp.minimum(mn, v), jnp.minimum(nmx, -v)

        return plsc.parallel_loop(0, _CH // _L, 1, unroll=8, carry=c)(step)

    init = (
        jnp.full((_L,), jnp.inf, jnp.float32),
        jnp.full((_L,), jnp.inf, jnp.float32),
    )
    cp = _scan_chunks(pos_hbm, w * (_NP // _NW), _NP // _NW // _CH, b0, b1, s0, s1, mm, init)
    cn = _scan_chunks(neg_hbm, w * (_NN // _NW), _NN // _NW // _CH, b0, b1, s0, s1, mm, init)
    ob[pl.ds(0, _L)] = cp[0]
    ob[pl.ds(16, _L)] = cp[1]
    ob[pl.ds(32, _L)] = cn[0]
    ob[pl.ds(48, _L)] = cn[1]
    pltpu.sync_copy(ob, out_hbm.at[pl.ds(w * 64, 64)])


@functools.partial(
    pl.kernel,
    out_type=jax.ShapeDtypeStruct((2 * _NW * 128,), jnp.float32),
    mesh=_MESH,
    compiler_params=pltpu.CompilerParams(needs_layout_passes=False),
    scratch_types=[
        pltpu.VMEM((_CH,), jnp.float32),
        pltpu.VMEM((_CH,), jnp.float32),
        pltpu.VMEM((_NW * 64,), jnp.float32),
        pltpu.VMEM((_L * _HP,), jnp.float32),
        pltpu.VMEM((_L * _HP,), jnp.float32),
        pltpu.VMEM((128,), jnp.float32),
        pltpu.SemaphoreType.DMA,
        pltpu.SemaphoreType.DMA,
    ],
)
def _hist_k(pos_hbm, neg_hbm, mm_hbm, out_hbm, b0, b1, mmb, hc, hf, ob, s0, s1):
    w = _wid()
    pltpu.sync_copy(mm_hbm, mmb)
    accs = [jnp.full((_L,), jnp.inf, jnp.float32) for _ in range(4)]
    for r in range(_NW):
        for q in range(4):
            accs[q] = jnp.minimum(accs[q], mmb[pl.ds(r * 64 + q * _L, _L)])
    minp = jnp.min(accs[0])
    maxp = -jnp.min(accs[1])
    minn = jnp.min(accs[2])
    maxn = -jnp.min(accs[3])
    lane = lax.iota(jnp.int32, _L)
    zeros = jnp.zeros((_L,), jnp.float32)

    def do_array(hbm, total, minv, maxv, row):
        for j in range(_HP):
            hc[pl.ds(j * _L, _L)] = zeros
            hf[pl.ds(j * _L, _L)] = zeros
        rng = jnp.full((_L,), maxv - minv, jnp.float32)
        scale = jnp.float32(1.0) / (rng * jnp.float32(_STEP))
        m2 = minv * scale
        ones = jnp.full((_L,), 1.0, jnp.float32)

        def hb(buf, c):
            def step(i):
                v = buf[pl.ds(i * _L, _L)]
                idx = v * scale - m2
                li = idx.astype(jnp.int32)
                frac = idx - li.astype(jnp.float32)
                bidx = li * _L + lane
                plsc.addupdate_scatter(hf, [bidx], frac)
                plsc.addupdate_scatter(hc, [bidx], ones, mask=frac > 0)

            plsc.parallel_loop(0, _CH // _L, 1, unroll=24)(step)
            return c

        _scan_chunks(hbm, w * (total // _NW), total // _NW // _CH, b0, b1, s0, s1, hb, 0)
        prev_f = zeros
        for blk in range(7):
            acc = zeros
            for b2 in range(_L):
                r0 = (blk * _L + b2) * _L
                s_c = _bfly_sum(hc[pl.ds(r0, _L)], lane)
                s_f = _bfly_sum(hf[pl.ds(r0, _L)], lane)
                acc = jnp.where(lane == b2, s_c - s_f + prev_f, acc)
                prev_f = s_f
            ob[pl.ds(blk * _L, _L)] = acc
        ob[pl.ds(7 * _L, _L)] = zeros
        pltpu.sync_copy(ob, out_hbm.at[pl.ds(row * 128, 128)])

    do_array(pos_hbm, _NP, minp, maxp, w)
    do_array(neg_hbm, _NN, minn, maxn, _NW + w)


@functools.partial(
    pl.kernel,
    out_type=jax.ShapeDtypeStruct((_L,), jnp.float32),
    mesh=_MESH,
    compiler_params=pltpu.CompilerParams(needs_layout_passes=False),
    scratch_types=[
        pltpu.VMEM((2 * _NW * 128,), jnp.float32),
        pltpu.VMEM((_L,), jnp.float32),
    ],
)
def _loss_k(parts_hbm, out_hbm, pb, ob):
    w = _wid()

    @pl.when(w == 0)
    def _():
        pltpu.sync_copy(parts_hbm, pb)
        lane = lax.iota(jnp.int32, _L)
        z7 = tuple(jnp.zeros((_L,), jnp.float32) for _ in range(7))

        def reduce_rows(r0, c):
            def step(r, acc):
                return tuple(
                    acc[j] + pb[pl.ds((r0 + r) * 128 + j * _L, _L)] for j in range(7)
                )

            return lax.fori_loop(0, _NW, step, c)

        hp = reduce_rows(0, z7)
        hn = reduce_rows(_NW, z7)
        loss = jnp.float32(0.0)
        carry = jnp.float32(0.0)
        for j in range(7):
            m = (j * _L + lane) <= _NB
            pj = jnp.where(m, hp[j] * jnp.float32(1.0 / _NP), 0.0)
            nj = jnp.where(m, hn[j] * jnp.float32(1.0 / _NN), 0.0)
            cj = plsc.cumsum(pj) + carry
            carry = carry + jnp.sum(pj)
            loss = loss + jnp.sum(cj * nj)
        ob[...] = jnp.where(lane == 0, loss, jnp.float32(0.0))
        pltpu.sync_copy(ob, out_hbm)


def kernel(sim_pos, sim_neg, n_bins):
    del n_bins
    mm = _minmax_k(sim_pos, sim_neg)
    parts = _hist_k(sim_pos, sim_neg, mm)
    return _loss_k(parts)[0]


# speedup vs baseline: 1.1622x; 1.1622x over previous
import functools

import numpy as np
import jax
import jax.numpy as jnp
from jax import lax
from jax.experimental import pallas as pl
from jax.experimental.pallas import tpu as pltpu
from jax.experimental.pallas import tpu_sc as plsc

_NB = 100
_NP = 4194304
_NN = 16777216
_NC, _NS, _L = 2, 16, 16
_NW = _NC * _NS
_CH = 32768
_HP = 112
_STEP = float(np.float32(1.0) / np.float32(_NB))

_MESH = plsc.VectorSubcoreMesh(
    core_axis_name="c", subcore_axis_name="s", num_cores=_NC, num_subcores=_NS
)


def _wid():
    return lax.axis_index("s") * _NC + lax.axis_index("c")


def _perm(v, idx):
    return lax.gather(
        v,
        idx[:, None],
        lax.GatherDimensionNumbers(
            offset_dims=(), collapsed_slice_dims=(0,), start_index_map=(0,)
        ),
        (1,),
        mode=lax.GatherScatterMode.PROMISE_IN_BOUNDS,
    )


def _bfly_sum(v, lane):
    for k in (1, 2, 4, 8):
        v = v + _perm(v, lane ^ k)
    return v


def _scan_chunks(hbm, base, nch, b0, b1, s0, s1, body, carry):
    pltpu.async_copy(hbm.at[pl.ds(base, _CH)], b0, s0)
    pltpu.async_copy(hbm.at[pl.ds(base + _CH, _CH)], b1, s1)

    def pair(p, c):
        off = base + 2 * p * _CH
        pltpu.make_async_copy(hbm.at[pl.ds(0, _CH)], b0, s0).wait()
        c = body(b0, c)

        @pl.when(2 * p + 2 < nch)
        def _():
            pltpu.async_copy(hbm.at[pl.ds(off + 2 * _CH, _CH)], b0, s0)

        pltpu.make_async_copy(hbm.at[pl.ds(0, _CH)], b1, s1).wait()
        c = body(b1, c)

        @pl.when(2 * p + 3 < nch)
        def _():
            pltpu.async_copy(hbm.at[pl.ds(off + 3 * _CH, _CH)], b1, s1)

        return c

    return lax.fori_loop(0, nch // 2, pair, carry)


@functools.partial(
    pl.kernel,
    out_type=jax.ShapeDtypeStruct((_NW * 64,), jnp.float32),
    mesh=_MESH,
    compiler_params=pltpu.CompilerParams(needs_layout_passes=False),
    scratch_types=[
        pltpu.VMEM((_CH,), jnp.float32),
        pltpu.VMEM((_CH,), jnp.float32),
        pltpu.VMEM((64,), jnp.float32),
        pltpu.SemaphoreType.DMA,
        pltpu.SemaphoreType.DMA,
    ],
)
def _minmax_k(pos_hbm, neg_hbm, out_hbm, b0, b1, ob, s0, s1):
    w = _wid()

    def mm(buf, c):
        def step(i, c2):
            mn, nmx = c2
            v = buf[pl.ds(i * _L, _L)]
            return jnp.minimum(mn, v), jnp.minimum(nmx, -v)

        return plsc.parallel_loop(0, _CH // _L, 1, unroll=8, carry=c)(step)

    init = (
        jnp.full((_L,), jnp.inf, jnp.float32),
        jnp.full((_L,), jnp.inf, jnp.float32),
    )
    cp = _scan_chunks(pos_hbm, w * (_NP // _NW), _NP // _NW // _CH, b0, b1, s0, s1, mm, init)
    cn = _scan_chunks(neg_hbm, w * (_NN // _NW), _NN // _NW // _CH, b0, b1, s0, s1, mm, init)
    ob[pl.ds(0, _L)] = cp[0]
    ob[pl.ds(16, _L)] = cp[1]
    ob[pl.ds(32, _L)] = cn[0]
    ob[pl.ds(48, _L)] = cn[1]
    pltpu.sync_copy(ob, out_hbm.at[pl.ds(w * 64, 64)])


@functools.partial(
    pl.kernel,
    out_type=jax.ShapeDtypeStruct((2 * _NW * 128,), jnp.float32),
    mesh=_MESH,
    compiler_params=pltpu.CompilerParams(needs_layout_passes=False),
    scratch_types=[
        pltpu.VMEM((_CH,), jnp.float32),
        pltpu.VMEM((_CH,), jnp.float32),
        pltpu.VMEM((_NW * 64,), jnp.float32),
        pltpu.VMEM((_L * _HP,), jnp.float32),
        pltpu.VMEM((_L * _HP,), jnp.float32),
        pltpu.VMEM((128,), jnp.float32),
        pltpu.SemaphoreType.DMA,
        pltpu.SemaphoreType.DMA,
    ],
)
def _hist_k(pos_hbm, neg_hbm, mm_hbm, out_hbm, b0, b1, mmb, hc, hf, ob, s0, s1):
    w = _wid()
    pltpu.sync_copy(mm_hbm, mmb)
    accs = [jnp.full((_L,), jnp.inf, jnp.float32) for _ in range(4)]
    for r in range(_NW):
        for q in range(4):
            accs[q] = jnp.minimum(accs[q], mmb[pl.ds(r * 64 + q * _L, _L)])
    minp = jnp.min(accs[0])
    maxp = -jnp.min(accs[1])
    minn = jnp.min(accs[2])
    maxn = -jnp.min(accs[3])
    lane = lax.iota(jnp.int32, _L)
    zeros = jnp.zeros((_L,), jnp.float32)

    def do_array(hbm, total, minv, maxv, row):
        for j in range(_HP):
            hc[pl.ds(j * _L, _L)] = zeros
            hf[pl.ds(j * _L, _L)] = zeros
        rng = jnp.full((_L,), maxv - minv, jnp.float32)
        scale = jnp.float32(1.0) / (rng * jnp.float32(_STEP))
        m2 = minv * scale
        ones = jnp.full((_L,), 1.0, jnp.float32)

        def hb(buf, c):
            def step(i):
                v = buf[pl.ds(i * _L, _L)]
                idx = v * scale - m2
                li = idx.astype(jnp.int32)
                frac = idx - li.astype(jnp.float32)
                bidx = li * _L + lane
                plsc.addupdate_scatter(hf, [bidx], frac)
                plsc.addupdate_scatter(hc, [bidx], ones)

            plsc.parallel_loop(0, _CH // _L, 1, unroll=16)(step)
            return c

        _scan_chunks(hbm, w * (total // _NW), total // _NW // _CH, b0, b1, s0, s1, hb, 0)
        prev_f = zeros
        for blk in range(7):
            acc = zeros
            for b2 in range(_L):
                r0 = (blk * _L + b2) * _L
                s_c = _bfly_sum(hc[pl.ds(r0, _L)], lane)
                s_f = _bfly_sum(hf[pl.ds(r0, _L)], lane)
                acc = jnp.where(lane == b2, s_c - s_f + prev_f, acc)
                prev_f = s_f
            ob[pl.ds(blk * _L, _L)] = acc
        ob[pl.ds(7 * _L, _L)] = zeros
        pltpu.sync_copy(ob, out_hbm.at[pl.ds(row * 128, 128)])

    do_array(pos_hbm, _NP, minp, maxp, w)
    do_array(neg_hbm, _NN, minn, maxn, _NW + w)


@functools.partial(
    pl.kernel,
    out_type=jax.ShapeDtypeStruct((_L,), jnp.float32),
    mesh=_MESH,
    compiler_params=pltpu.CompilerParams(needs_layout_passes=False),
    scratch_types=[
        pltpu.VMEM((2 * _NW * 128,), jnp.float32),
        pltpu.VMEM((_L,), jnp.float32),
    ],
)
def _loss_k(parts_hbm, out_hbm, pb, ob):
    w = _wid()

    @pl.when(w == 0)
    def _():
        pltpu.sync_copy(parts_hbm, pb)
        lane = lax.iota(jnp.int32, _L)
        z7 = tuple(jnp.zeros((_L,), jnp.float32) for _ in range(7))

        def reduce_rows(r0, c):
            def step(r, acc):
                return tuple(
                    acc[j] + pb[pl.ds((r0 + r) * 128 + j * _L, _L)] for j in range(7)
                )

            return lax.fori_loop(0, _NW, step, c)

        hp = reduce_rows(0, z7)
        hn = reduce_rows(_NW, z7)
        loss = jnp.float32(0.0)
        carry = jnp.float32(0.0)
        for j in range(7):
            m = (j * _L + lane) <= _NB
            pj = jnp.where(m, hp[j] * jnp.float32(1.0 / _NP), 0.0)
            nj = jnp.where(m, hn[j] * jnp.float32(1.0 / _NN), 0.0)
            cj = plsc.cumsum(pj) + carry
            carry = carry + jnp.sum(pj)
            loss = loss + jnp.sum(cj * nj)
        ob[...] = jnp.where(lane == 0, loss, jnp.float32(0.0))
        pltpu.sync_copy(ob, out_hbm)


def kernel(sim_pos, sim_neg, n_bins):
    del n_bins
    mm = _minmax_k(sim_pos, sim_neg)
    parts = _hist_k(sim_pos, sim_neg, mm)
    return _loss_k(parts)[0]
